# baseline (speedup 1.0000x reference)
import jax
import jax.numpy as jnp
from jax import lax
from jax.experimental import pallas as pl
from jax.experimental.pallas import tpu as pltpu


_NEG_BIG = -1e30


def _ceil_to(n, m):
    return ((n + m - 1) // m) * m


def _batch_tiling(B, tb_max=1024):
    b8 = _ceil_to(B, 8)
    tb = min(tb_max, b8)
    if b8 >= 16 and _ceil_to(B, tb) // tb < 2:
        tb = max(8, _ceil_to(b8 // 2, 8))
    b_pad = _ceil_to(B, tb)
    return tb, b_pad, b_pad // tb


def _make_stats_kernel(TB, D, Tpad):
    def _body(x_ref, lp_ref, s1_ref, s2_ref, ps_ref):
        xv = x_ref[...]
        s1_ref[...] = jnp.sum(xv.reshape(TB // 8, 8, D), axis=0)
        s2_ref[...] = jnp.sum((xv * xv).reshape(TB // 8, 8, D), axis=0)
        ps_ref[...] = jnp.sum(lp_ref[...].reshape(TB // 8, 8, Tpad), axis=0)

    return _body


def _make_phi_kernel(B, TB, Tpad):
    def _body(x_ref, lp_ref, wT_ref, nm_ref, wr_ref, nr_ref, bias_ref,
              pneg_ref, s1_ref, s2_ref, ps_ref, phi_ref, lik_ref):
        blk = pl.program_id(0)

        s1 = jnp.sum(s1_ref[...], axis=0, keepdims=True)
        s2 = jnp.sum(s2_ref[...], axis=0, keepdims=True)
        n_t_pi = jnp.sum(ps_ref[...], axis=0, keepdims=True)
        quad_sum = (jnp.dot(s2, wr_ref[...], precision=lax.Precision.HIGHEST,
                            preferred_element_type=jnp.float32)
                    + jnp.dot(s1, nr_ref[...], precision=lax.Precision.HIGHEST,
                              preferred_element_type=jnp.float32))
        bias = bias_ref[...]
        den = (B * bias - quad_sum) + n_t_pi
        den = jnp.where(jnp.abs(den) < 1e-12,
                        jnp.where(den >= 0, 1e-12, -1e-12), den)
        mixv = n_t_pi / den
        bm = bias * mixv + pneg_ref[...]
        om = 1.0 - mixv

        xv = x_ref[...]
        quad_m = jnp.dot(xv * xv, wT_ref[...] * mixv,
                         preferred_element_type=jnp.float32)
        quad_m = quad_m + jnp.dot(xv, nm_ref[...] * mixv,
                                  preferred_element_type=jnp.float32)
        mklg = bm - quad_m
        logits = mklg + om * lp_ref[...]

        m = jnp.max(logits, axis=1, keepdims=True)
        e = jnp.exp(logits - m)
        phi = e * (1.0 / jnp.sum(e, axis=1, keepdims=True))
        phi_ref[...] = phi

        row = lax.broadcasted_iota(jnp.int32, (TB, 1), 0)
        valid = (row < (B - blk * TB)).astype(jnp.float32)
        contrib = phi * mklg * valid
        lik_ref[...] = jnp.sum(contrib.reshape(TB // 8, 8, Tpad), axis=0)

    return _body


def kernel(x, mu, rho, log_pi):
    x = x.astype(jnp.float32)
    mu = mu.astype(jnp.float32)
    rho = rho.astype(jnp.float32)
    log_pi = log_pi.astype(jnp.float32)

    B, D = x.shape
    T = mu.shape[0]
    Tpad = _ceil_to(max(T, 128), 128)
    padT = Tpad - T
    TB, B_pad, nblocks = _batch_tiling(B)

    std = jax.nn.softplus(rho)
    w = 0.5 / (std * std)
    wT = w.T
    nmwT = (-2.0 * mu * w).T
    bias = 0.5 * D - jnp.sum(mu * mu * w, axis=1)

    x_p = jnp.pad(x, ((0, B_pad - B), (0, 0)))
    logpi_p = jnp.pad(log_pi, ((0, B_pad - B), (0, padT)))

    cparams = pltpu.CompilerParams(
        dimension_semantics=("parallel",),
        vmem_limit_bytes=32 * 1024 * 1024,
    )

    s1_parts, s2_parts, ps_parts = pl.pallas_call(
        _make_stats_kernel(TB, D, Tpad),
        grid=(nblocks,),
        in_specs=[
            pl.BlockSpec((TB, D), lambda i: (i, 0)),
            pl.BlockSpec((TB, Tpad), lambda i: (i, 0)),
        ],
        out_specs=(
            pl.BlockSpec((8, D), lambda i: (i, 0)),
            pl.BlockSpec((8, D), lambda i: (i, 0)),
            pl.BlockSpec((8, Tpad), lambda i: (i, 0)),
        ),
        out_shape=(
            jax.ShapeDtypeStruct((8 * nblocks, D), jnp.float32),
            jax.ShapeDtypeStruct((8 * nblocks, D), jnp.float32),
            jax.ShapeDtypeStruct((8 * nblocks, Tpad), jnp.float32),
        ),
        compiler_params=cparams,
    )(x_p, logpi_p)

    wT_p = jnp.pad(wT, ((0, 0), (0, padT)))
    nmwT_p = jnp.pad(nmwT, ((0, 0), (0, padT)))
    wr_p = wT_p.astype(jnp.bfloat16).astype(jnp.float32)
    nr_p = nmwT_p.astype(jnp.bfloat16).astype(jnp.float32)
    bias_p = jnp.pad(bias[None, :], ((0, 0), (0, padT)))
    pneg_p = jnp.pad(jnp.zeros((1, T), jnp.float32), ((0, 0), (0, padT)),
                     constant_values=_NEG_BIG)

    phi_p, lik_parts = pl.pallas_call(
        _make_phi_kernel(B, TB, Tpad),
        grid=(nblocks,),
        in_specs=[
            pl.BlockSpec((TB, D), lambda i: (i, 0)),
            pl.BlockSpec((TB, Tpad), lambda i: (i, 0)),
            pl.BlockSpec((D, Tpad), lambda i: (0, 0)),
            pl.BlockSpec((D, Tpad), lambda i: (0, 0)),
            pl.BlockSpec((D, Tpad), lambda i: (0, 0)),
            pl.BlockSpec((D, Tpad), lambda i: (0, 0)),
            pl.BlockSpec((1, Tpad), lambda i: (0, 0)),
            pl.BlockSpec((1, Tpad), lambda i: (0, 0)),
            pl.BlockSpec((8 * nblocks, D), lambda i: (0, 0)),
            pl.BlockSpec((8 * nblocks, D), lambda i: (0, 0)),
            pl.BlockSpec((8 * nblocks, Tpad), lambda i: (0, 0)),
        ],
        out_specs=(
            pl.BlockSpec((TB, Tpad), lambda i: (i, 0)),
            pl.BlockSpec((8, Tpad), lambda i: (i, 0)),
        ),
        out_shape=(
            jax.ShapeDtypeStruct((B_pad, Tpad), jnp.float32),
            jax.ShapeDtypeStruct((8 * nblocks, Tpad), jnp.float32),
        ),
        compiler_params=cparams,
    )(x_p, logpi_p, wT_p, nmwT_p, wr_p, nr_p, bias_p, pneg_p,
      s1_parts, s2_parts, ps_parts)

    likelihood = jnp.sum(lik_parts) / float(B)
    phi_new = phi_p[:B, :T]
    return likelihood, phi_new


# speedup vs baseline: 1.3060x; 1.3060x over previous
import jax
import jax.numpy as jnp
from jax import lax
from jax.experimental import pallas as pl
from jax.experimental.pallas import tpu as pltpu


_NEG_BIG = -1e30


def _ceil_to(n, m):
    return ((n + m - 1) // m) * m


def _batch_tiling(B, tb_max=2048):
    b8 = _ceil_to(B, 8)
    tb = min(tb_max, b8)
    if b8 >= 16 and _ceil_to(B, tb) // tb < 2:
        tb = max(8, _ceil_to(b8 // 2, 8))
    b_pad = _ceil_to(B, tb)
    return tb, b_pad, b_pad // tb


def _make_stats_kernel(TB, D, Tpad):
    def _body(x_ref, lp_ref, s1_ref, s2_ref, ps_ref):
        xv = x_ref[...]
        s1_ref[...] = jnp.sum(xv.reshape(TB // 8, 8, D), axis=0)
        s2_ref[...] = jnp.sum((xv * xv).reshape(TB // 8, 8, D), axis=0)
        ps_ref[...] = jnp.sum(lp_ref[...].reshape(TB // 8, 8, Tpad), axis=0)

    return _body


def _make_phi_kernel(B, TB, Tpad):
    def _body(x_ref, lp_ref, wm_ref, nm_ref, bm_ref, om_ref, phi_ref, lik_ref):
        blk = pl.program_id(0)

        xv = x_ref[...]
        quad_m = jnp.dot(xv * xv, wm_ref[...],
                         preferred_element_type=jnp.float32)
        quad_m = quad_m + jnp.dot(xv, nm_ref[...],
                                  preferred_element_type=jnp.float32)
        mklg = bm_ref[...] - quad_m
        logits = mklg + om_ref[...] * lp_ref[...]

        m = jnp.max(logits, axis=1, keepdims=True)
        e = jnp.exp(logits - m)
        phi = e * (1.0 / jnp.sum(e, axis=1, keepdims=True))
        phi_ref[...] = phi

        row = lax.broadcasted_iota(jnp.int32, (TB, 1), 0)
        valid = (row < (B - blk * TB)).astype(jnp.float32)
        contrib = phi * mklg * valid
        lik_ref[...] = jnp.sum(contrib.reshape(TB // 8, 8, Tpad), axis=0)

    return _body


def kernel(x, mu, rho, log_pi):
    x = x.astype(jnp.float32)
    mu = mu.astype(jnp.float32)
    rho = rho.astype(jnp.float32)
    log_pi = log_pi.astype(jnp.float32)

    B, D = x.shape
    T = mu.shape[0]
    Tpad = _ceil_to(max(T, 128), 128)
    padT = Tpad - T
    TB, B_pad, nblocks = _batch_tiling(B)

    std = jax.nn.softplus(rho)
    w = 0.5 / (std * std)
    wT = w.T
    nmwT = (-2.0 * mu * w).T
    bias = 0.5 * D - jnp.sum(mu * mu * w, axis=1)

    x_p = jnp.pad(x, ((0, B_pad - B), (0, 0)))
    logpi_p = jnp.pad(log_pi, ((0, B_pad - B), (0, padT)))

    cparams = pltpu.CompilerParams(
        dimension_semantics=("parallel",),
        vmem_limit_bytes=32 * 1024 * 1024,
    )

    s1_parts, s2_parts, ps_parts = pl.pallas_call(
        _make_stats_kernel(TB, D, Tpad),
        grid=(nblocks,),
        in_specs=[
            pl.BlockSpec((TB, D), lambda i: (i, 0)),
            pl.BlockSpec((TB, Tpad), lambda i: (i, 0)),
        ],
        out_specs=(
            pl.BlockSpec((8, D), lambda i: (i, 0)),
            pl.BlockSpec((8, D), lambda i: (i, 0)),
            pl.BlockSpec((8, Tpad), lambda i: (i, 0)),
        ),
        out_shape=(
            jax.ShapeDtypeStruct((8 * nblocks, D), jnp.float32),
            jax.ShapeDtypeStruct((8 * nblocks, D), jnp.float32),
            jax.ShapeDtypeStruct((8 * nblocks, Tpad), jnp.float32),
        ),
        compiler_params=cparams,
    )(x_p, logpi_p)

    s1 = jnp.sum(s1_parts, axis=0)
    s2 = jnp.sum(s2_parts, axis=0)
    n_t_pi = jnp.sum(ps_parts, axis=0)[:T]

    wT_r = wT.astype(jnp.bfloat16).astype(jnp.float32)
    nmwT_r = nmwT.astype(jnp.bfloat16).astype(jnp.float32)
    quad_sum = (jnp.dot(s2, wT_r, precision=lax.Precision.HIGHEST)
                + jnp.dot(s1, nmwT_r, precision=lax.Precision.HIGHEST))

    n_t_gaussian = B * bias - quad_sum
    den = n_t_gaussian + n_t_pi
    den = jnp.where(jnp.abs(den) < 1e-12,
                    jnp.where(den >= 0, 1e-12, -1e-12), den)
    mix = n_t_pi / den

    wm_p = jnp.pad(wT * mix[None, :], ((0, 0), (0, padT)))
    nm_p = jnp.pad(nmwT * mix[None, :], ((0, 0), (0, padT)))
    bm_p = jnp.pad((bias * mix)[None, :], ((0, 0), (0, padT)),
                   constant_values=_NEG_BIG)
    om_p = jnp.pad((1.0 - mix)[None, :], ((0, 0), (0, padT)))

    phi_p, lik_parts = pl.pallas_call(
        _make_phi_kernel(B, TB, Tpad),
        grid=(nblocks,),
        in_specs=[
            pl.BlockSpec((TB, D), lambda i: (i, 0)),
            pl.BlockSpec((TB, Tpad), lambda i: (i, 0)),
            pl.BlockSpec((D, Tpad), lambda i: (0, 0)),
            pl.BlockSpec((D, Tpad), lambda i: (0, 0)),
            pl.BlockSpec((1, Tpad), lambda i: (0, 0)),
            pl.BlockSpec((1, Tpad), lambda i: (0, 0)),
        ],
        out_specs=(
            pl.BlockSpec((TB, Tpad), lambda i: (i, 0)),
            pl.BlockSpec((8, Tpad), lambda i: (i, 0)),
        ),
        out_shape=(
            jax.ShapeDtypeStruct((B_pad, Tpad), jnp.float32),
            jax.ShapeDtypeStruct((8 * nblocks, Tpad), jnp.float32),
        ),
        compiler_params=cparams,
    )(x_p, logpi_p, wm_p, nm_p, bm_p, om_p)

    likelihood = jnp.sum(lik_parts) / float(B)
    phi_new = phi_p[:B, :T]
    return likelihood, phi_new
